# Initial kernel scaffold; baseline (speedup 1.0000x reference)
import jax
import jax.numpy as jnp
from jax.experimental import pallas as pl


def kernel(img, flow, scale):
    raise NotImplementedError("write your pallas kernel here")


# speedup vs baseline: 15.3386x; 15.3386x over previous
import functools

import jax
import jax.numpy as jnp
from jax import lax
from jax.experimental import pallas as pl
from jax.experimental.pallas import tpu as pltpu
from jax.experimental.pallas import tpu_sc as plsc

SCALE_CONST = 4
NC, NS, L = 2, 16, 16
NW = NC * NS


def _build(batch, chans, h, w, chunk, interpret=False):
    n = h * w
    oh, ow = h * SCALE_CONST, w * SCALE_CONST
    rb = oh // NW
    band = chans * rb * ow
    vpc = chunk // L
    nchunk = n // chunk
    wshift = w.bit_length() - 1
    wmask = w - 1
    assert (1 << wshift) == w and n % chunk == 0 and oh % NW == 0

    def body(fy_hbm, fx_hbm, img_hbm, scl_hbm, out_hbm,
             acc, fyb, fxb, cbufs, sclb, flagb):

        def any_true(mvec):
            return plsc.all_reduce_population_count(mvec)[0] > 0
        cid = lax.axis_index("c")
        sid = lax.axis_index("s")
        wid = sid * NC + cid
        lo = wid * rb

        pltpu.sync_copy(scl_hbm, sclb)
        scale_v = sclb[...]

        iota = lax.iota(jnp.int32, L)
        lo_f = lo.astype(jnp.float32)
        band_lo = jnp.full((L,), lo_f - 1.0, jnp.float32)
        band_hi = jnp.full((L,), lo_f + rb, jnp.float32)
        zeros = jnp.zeros((L,), jnp.float32)

        def per_batch(b, carry):
            def zero_loop(i, c):
                off = i * (8 * L)
                for k in range(8):
                    acc[pl.ds(off + k * L, L)] = zeros
                return c
            lax.fori_loop(0, band // (8 * L), zero_loop, 0)

            def chunk_loop(j, c):
                base = j * chunk
                hbase = b * n + base
                pltpu.sync_copy(fy_hbm.at[pl.ds(hbase, chunk)], fyb)

                def scan(i, hitv):
                    pi = base + i * L + iota
                    gy = (pi >> wshift).astype(jnp.float32)
                    y = (gy + fyb[pl.ds(i * L, L)]) * scale_v
                    hit = (y >= band_lo) & (y < band_hi)
                    return hitv | jnp.where(hit, 1, 0)
                hitv = lax.fori_loop(0, vpc, scan,
                                     jnp.zeros((L,), jnp.int32))
                any_hit = any_true(hitv != 0)

                @pl.when(any_hit)
                def _process():
                    pltpu.sync_copy(fx_hbm.at[pl.ds(hbase, chunk)], fxb)
                    ibase = b * chans * n + base
                    for ch in range(chans):
                        pltpu.sync_copy(
                            img_hbm.at[pl.ds(ibase + ch * n, chunk)],
                            cbufs[ch])

                    def proc(i, c2):
                        sl = pl.ds(i * L, L)
                        pi = base + i * L + iota
                        gy = (pi >> wshift).astype(jnp.float32)
                        y = (gy + fyb[sl]) * scale_v
                        hit = (y >= band_lo) & (y < band_hi)
                        vreg_hit = any_true(hit)

                        @pl.when(vreg_hit)
                        def _splat():
                            gx = (pi & wmask).astype(jnp.float32)
                            x = (gx + fxb[sl]) * scale_v
                            xt = x.astype(jnp.int32)
                            x0 = xt - jnp.where(
                                xt.astype(jnp.float32) > x, 1, 0)
                            yt = y.astype(jnp.int32)
                            y0 = yt - jnp.where(
                                yt.astype(jnp.float32) > y, 1, 0)
                            wx1 = x - x0.astype(jnp.float32)
                            wx0 = 1.0 - wx1
                            wy1 = y - y0.astype(jnp.float32)
                            wy0 = 1.0 - wy1
                            vals = [cb[sl] for cb in cbufs]
                            ly = y0 - lo
                            x1 = x0 + 1
                            for lyv, wy in ((ly, wy0), (ly + 1, wy1)):
                                my = (lyv >= 0) & (lyv < rb)
                                rowb = lyv * ow
                                for xv, wx in ((x0, wx0), (x1, wx1)):
                                    m = my & (xv >= 0) & (xv < ow)
                                    wgt = wy * wx
                                    idx = jnp.where(m, rowb + xv, 0)
                                    for ch in range(chans):
                                        plsc.addupdate_scatter(
                                            acc, [idx + ch * rb * ow],
                                            vals[ch] * wgt, mask=m)
                        return c2
                    lax.fori_loop(0, vpc, proc, 0)
                return c
            lax.fori_loop(0, nchunk, chunk_loop, 0)

            obase = b * chans * oh * ow + lo * ow
            for ch in range(chans):
                pltpu.sync_copy(
                    acc.at[pl.ds(ch * rb * ow, rb * ow)],
                    out_hbm.at[pl.ds(obase + ch * oh * ow, rb * ow)])
            return carry

        lax.fori_loop(0, batch, per_batch, 0)

    mesh = plsc.VectorSubcoreMesh(core_axis_name="c", subcore_axis_name="s",
                                  num_cores=NC, num_subcores=NS)

    def wrapped(fy_hbm, fx_hbm, img_hbm, scl_hbm, out_hbm,
                acc, fyb, fxb, c0, c1, c2, sclb, flagb):
        return body(fy_hbm, fx_hbm, img_hbm, scl_hbm, out_hbm,
                    acc, fyb, fxb, [c0, c1, c2], sclb, flagb)

    @jax.jit
    def warp(fy, fx, imgf, scl):
        return pl.kernel(
            wrapped,
            out_type=jax.ShapeDtypeStruct((batch * chans * oh * ow,),
                                          jnp.float32),
            mesh=mesh,
            interpret=interpret,
            compiler_params=pltpu.CompilerParams(needs_layout_passes=False),
            scratch_types=[
                pltpu.VMEM((band,), jnp.float32),
                pltpu.VMEM((chunk,), jnp.float32),
                pltpu.VMEM((chunk,), jnp.float32),
                pltpu.VMEM((chunk,), jnp.float32),
                pltpu.VMEM((chunk,), jnp.float32),
                pltpu.VMEM((chunk,), jnp.float32),
                pltpu.VMEM((L,), jnp.float32),
                pltpu.VMEM((L,), jnp.int32),
            ],
        )(fy, fx, imgf, scl)

    def run(img, flow, scale):
        fy = flow[:, 1, :, :].reshape(batch * n)
        fx = flow[:, 0, :, :].reshape(batch * n)
        imgf = img.reshape(batch * chans * n)
        scl = jnp.full((L,), scale, jnp.float32)
        out = warp(fy, fx, imgf, scl)
        return out.reshape(batch, chans, oh, ow)

    return run


_run = _build(8, 3, 256, 256, 2048)


def kernel(img, flow, scale):
    return _run(img, flow, scale)


# speedup vs baseline: 31.0827x; 2.0264x over previous
import functools

import jax
import jax.numpy as jnp
from jax import lax
from jax.experimental import pallas as pl
from jax.experimental.pallas import tpu as pltpu
from jax.experimental.pallas import tpu_sc as plsc

SCALE_CONST = 4
NC, NS, L = 2, 16, 16
NW = NC * NS


def _build(batch, chans, h, w, chunk, interpret=False):
    n = h * w
    oh, ow = h * SCALE_CONST, w * SCALE_CONST
    rb = oh // NW
    band = chans * rb * ow
    vpc = chunk // L
    nchunk = n // chunk
    npair = batch * nchunk
    ppt = npair // NS
    wshift = w.bit_length() - 1
    wmask = w - 1
    assert (1 << wshift) == w and n % chunk == 0 and oh % NW == 0
    assert npair % NS == 0
    jshift = nchunk.bit_length() - 1
    assert (1 << jshift) == nchunk

    def body(fy_hbm, fx_hbm, img_hbm, scl_hbm, out_hbm,
             acc, fyb, fxb, cbufs, sclb, minb, maxb, mint, maxt,
             shmin, shmax, sem, osem):
        cid = lax.axis_index("c")
        sid = lax.axis_index("s")
        wid = sid * NC + cid
        lo = wid * rb

        def any_true(mvec):
            return plsc.all_reduce_population_count(mvec)[0] > 0

        pltpu.sync_copy(scl_hbm, sclb)
        scale_v = sclb[...]

        iota = lax.iota(jnp.int32, L)
        lo_f = lo.astype(jnp.float32)
        band_lo = jnp.full((L,), lo_f - 1.0, jnp.float32)
        band_hi = jnp.full((L,), lo_f + rb, jnp.float32)
        zeros = jnp.zeros((L,), jnp.float32)

        def range_pair(i, c):
            k = sid * ppt + i
            b = k >> jshift
            j = k - (b << jshift)
            pltpu.sync_copy(fy_hbm.at[pl.ds(b * n + j * chunk, chunk)], fyb)

            def mm(v, carry):
                mn, mx = carry
                pi = j * chunk + v * L + iota
                gy = (pi >> wshift).astype(jnp.float32)
                y = (gy + fyb[pl.ds(v * L, L)]) * scale_v
                return jnp.minimum(mn, y), jnp.maximum(mx, y)
            mn, mx = lax.fori_loop(
                0, vpc, mm,
                (jnp.full((L,), 3.0e38, jnp.float32),
                 jnp.full((L,), -3.0e38, jnp.float32)))
            minb[pl.ds(i * L, L)] = mn
            maxb[pl.ds(i * L, L)] = mx
            return c
        lax.fori_loop(0, ppt, range_pair, 0)
        pltpu.sync_copy(minb, shmin.at[pl.ds(sid * ppt * L, ppt * L)])
        pltpu.sync_copy(maxb, shmax.at[pl.ds(sid * ppt * L, ppt * L)])
        plsc.subcore_barrier()
        pltpu.sync_copy(shmin, mint)
        pltpu.sync_copy(shmax, maxt)

        def zero_range(c0, nwords):
            def zloop(i, c):
                off = c0 + i * (8 * L)
                for k in range(8):
                    acc[pl.ds(off + k * L, L)] = zeros
                return c
            lax.fori_loop(0, nwords // (8 * L), zloop, 0)

        zero_range(0, band)

        def per_batch(b, carry):
            def chunk_loop(j, c):
                k16 = (b * nchunk + j) * L
                cmin = mint[pl.ds(k16, L)]
                cmax = maxt[pl.ds(k16, L)]
                hit = (cmax >= band_lo) & (cmin < band_hi)

                @pl.when(any_true(hit))
                def _process():
                    base = j * chunk
                    hbase = b * n + base
                    ibase = b * chans * n + base
                    cps = [
                        pltpu.async_copy(
                            fy_hbm.at[pl.ds(hbase, chunk)], fyb, sem),
                        pltpu.async_copy(
                            fx_hbm.at[pl.ds(hbase, chunk)], fxb, sem),
                    ] + [
                        pltpu.async_copy(
                            img_hbm.at[pl.ds(ibase + ch * n, chunk)],
                            cbufs[ch], sem)
                        for ch in range(chans)
                    ]
                    for cp in cps:
                        cp.wait()

                    def proc(i, c2):
                        sl = pl.ds(i * L, L)
                        pi = base + i * L + iota
                        gy = (pi >> wshift).astype(jnp.float32)
                        y = (gy + fyb[sl]) * scale_v
                        hitv = (y >= band_lo) & (y < band_hi)

                        @pl.when(any_true(hitv))
                        def _splat():
                            gx = (pi & wmask).astype(jnp.float32)
                            x = (gx + fxb[sl]) * scale_v
                            xt = x.astype(jnp.int32)
                            x0 = xt - jnp.where(
                                xt.astype(jnp.float32) > x, 1, 0)
                            yt = y.astype(jnp.int32)
                            y0 = yt - jnp.where(
                                yt.astype(jnp.float32) > y, 1, 0)
                            wx1 = x - x0.astype(jnp.float32)
                            wx0 = 1.0 - wx1
                            wy1 = y - y0.astype(jnp.float32)
                            wy0 = 1.0 - wy1
                            vals = [cb[sl] for cb in cbufs]
                            ly = y0 - lo
                            x1 = x0 + 1
                            for lyv, wy in ((ly, wy0), (ly + 1, wy1)):
                                my = (lyv >= 0) & (lyv < rb)
                                rowb = lyv * ow
                                for xv, wx in ((x0, wx0), (x1, wx1)):
                                    m = my & (xv >= 0) & (xv < ow)
                                    wgt = wy * wx
                                    idx = jnp.where(m, rowb + xv, 0)
                                    for ch in range(chans):
                                        plsc.addupdate_scatter(
                                            acc, [idx + ch * rb * ow],
                                            vals[ch] * wgt, mask=m)
                        return c2
                    lax.fori_loop(0, vpc, proc, 0)
                return c
            lax.fori_loop(0, nchunk, chunk_loop, 0)

            obase = b * chans * oh * ow + lo * ow
            flushes = [
                pltpu.async_copy(
                    acc.at[pl.ds(ch * rb * ow, rb * ow)],
                    out_hbm.at[pl.ds(obase + ch * oh * ow, rb * ow)],
                    osem)
                for ch in range(chans)
            ]
            last = b == batch - 1
            for ch in range(chans):
                flushes[ch].wait()

                @pl.when(jnp.logical_not(last))
                def _rezero():
                    zero_range(ch * rb * ow, rb * ow)
            return carry

        lax.fori_loop(0, batch, per_batch, 0)

    mesh = plsc.VectorSubcoreMesh(core_axis_name="c", subcore_axis_name="s",
                                  num_cores=NC, num_subcores=NS)

    def wrapped(fy_hbm, fx_hbm, img_hbm, scl_hbm, out_hbm,
                acc, fyb, fxb, c0, c1, c2, sclb,
                minb, maxb, mint, maxt, shmin, shmax, sem, osem):
        return body(fy_hbm, fx_hbm, img_hbm, scl_hbm, out_hbm,
                    acc, fyb, fxb, [c0, c1, c2], sclb,
                    minb, maxb, mint, maxt, shmin, shmax, sem, osem)

    @jax.jit
    def warp(fy, fx, imgf, scl):
        return pl.kernel(
            wrapped,
            out_type=jax.ShapeDtypeStruct((batch * chans * oh * ow,),
                                          jnp.float32),
            mesh=mesh,
            interpret=interpret,
            compiler_params=pltpu.CompilerParams(needs_layout_passes=False),
            scratch_types=[
                pltpu.VMEM((band,), jnp.float32),
                pltpu.VMEM((chunk,), jnp.float32),
                pltpu.VMEM((chunk,), jnp.float32),
                pltpu.VMEM((chunk,), jnp.float32),
                pltpu.VMEM((chunk,), jnp.float32),
                pltpu.VMEM((chunk,), jnp.float32),
                pltpu.VMEM((L,), jnp.float32),
                pltpu.VMEM((ppt * L,), jnp.float32),
                pltpu.VMEM((ppt * L,), jnp.float32),
                pltpu.VMEM((npair * L,), jnp.float32),
                pltpu.VMEM((npair * L,), jnp.float32),
                pltpu.VMEM_SHARED((npair * L,), jnp.float32),
                pltpu.VMEM_SHARED((npair * L,), jnp.float32),
                pltpu.SemaphoreType.DMA,
                pltpu.SemaphoreType.DMA,
            ],
        )(fy, fx, imgf, scl)

    def run(img, flow, scale):
        fy = flow[:, 1, :, :].reshape(batch * n)
        fx = flow[:, 0, :, :].reshape(batch * n)
        imgf = img.reshape(batch * chans * n)
        scl = jnp.full((L,), scale, jnp.float32)
        out = warp(fy, fx, imgf, scl)
        return out.reshape(batch, chans, oh, ow)

    return run


_run = _build(8, 3, 256, 256, 2048)


def kernel(img, flow, scale):
    return _run(img, flow, scale)


# speedup vs baseline: 31.1435x; 1.0020x over previous
import functools

import jax
import jax.numpy as jnp
from jax import lax
from jax.experimental import pallas as pl
from jax.experimental.pallas import tpu as pltpu
from jax.experimental.pallas import tpu_sc as plsc

SCALE_CONST = 4
NC, NS, L = 2, 16, 16
NW = NC * NS


def _build(batch, chans, h, w, chunk, interpret=False):
    n = h * w
    oh, ow = h * SCALE_CONST, w * SCALE_CONST
    rb = oh // NW
    band = chans * rb * ow
    vpc = chunk // L
    nchunk = n // chunk
    npair = batch * nchunk
    ppt = npair // NS
    wshift = w.bit_length() - 1
    wmask = w - 1
    assert (1 << wshift) == w and n % chunk == 0 and oh % NW == 0
    assert npair % NS == 0
    jshift = nchunk.bit_length() - 1
    assert (1 << jshift) == nchunk

    def body(fy_hbm, fx_hbm, img_hbm, scl_hbm, out_hbm,
             acc, fyb, fxb, cbufs, sclb, minb, maxb, mint, maxt,
             shmin, shmax, sem, osem):
        cid = lax.axis_index("c")
        sid = lax.axis_index("s")
        wid = sid * NC + cid
        lo = wid * rb

        def any_true(mvec):
            return plsc.all_reduce_population_count(mvec)[0] > 0

        pltpu.sync_copy(scl_hbm, sclb)
        scale_v = sclb[...]

        iota = lax.iota(jnp.int32, L)
        lo_f = lo.astype(jnp.float32)
        band_lo = jnp.full((L,), lo_f - 1.0, jnp.float32)
        band_hi = jnp.full((L,), lo_f + rb, jnp.float32)
        zeros = jnp.zeros((L,), jnp.float32)

        scope = jax.named_scope

        def range_pair(i, c):
            k = sid * ppt + i
            b = k >> jshift
            j = k - (b << jshift)
            pltpu.sync_copy(fy_hbm.at[pl.ds(b * n + j * chunk, chunk)], fyb)

            def mm(v, carry):
                mn, mx = carry
                pi = j * chunk + v * L + iota
                gy = (pi >> wshift).astype(jnp.float32)
                y = (gy + fyb[pl.ds(v * L, L)]) * scale_v
                return jnp.minimum(mn, y), jnp.maximum(mx, y)
            mn, mx = lax.fori_loop(
                0, vpc, mm,
                (jnp.full((L,), 3.0e38, jnp.float32),
                 jnp.full((L,), -3.0e38, jnp.float32)))
            minb[pl.ds(i * L, L)] = mn
            maxb[pl.ds(i * L, L)] = mx
            return c
        with scope("phase0_range"):
            lax.fori_loop(0, ppt, range_pair, 0)
            pltpu.sync_copy(minb, shmin.at[pl.ds(sid * ppt * L, ppt * L)])
            pltpu.sync_copy(maxb, shmax.at[pl.ds(sid * ppt * L, ppt * L)])
            plsc.subcore_barrier()
            pltpu.sync_copy(shmin, mint)
            pltpu.sync_copy(shmax, maxt)

        def zero_range(c0, nwords):
            def zloop(i, c):
                off = c0 + i * (8 * L)
                for k in range(8):
                    acc[pl.ds(off + k * L, L)] = zeros
                return c
            lax.fori_loop(0, nwords // (8 * L), zloop, 0)

        with scope("zero_init"):
            zero_range(0, band)

        def per_batch(b, carry):
            def chunk_loop(j, c):
                k16 = (b * nchunk + j) * L
                cmin = mint[pl.ds(k16, L)]
                cmax = maxt[pl.ds(k16, L)]
                hit = (cmax >= band_lo) & (cmin < band_hi)

                @pl.when(any_true(hit))
                def _process():
                    base = j * chunk
                    hbase = b * n + base
                    ibase = b * chans * n + base
                    cps = [
                        pltpu.async_copy(
                            fy_hbm.at[pl.ds(hbase, chunk)], fyb, sem),
                        pltpu.async_copy(
                            fx_hbm.at[pl.ds(hbase, chunk)], fxb, sem),
                    ] + [
                        pltpu.async_copy(
                            img_hbm.at[pl.ds(ibase + ch * n, chunk)],
                            cbufs[ch], sem)
                        for ch in range(chans)
                    ]
                    with scope("fetch_wait"):
                        for cp in cps:
                            cp.wait()

                    def proc(i, c2):
                        sl = pl.ds(i * L, L)
                        pi = base + i * L + iota
                        gy = (pi >> wshift).astype(jnp.float32)
                        y = (gy + fyb[sl]) * scale_v
                        hitv = (y >= band_lo) & (y < band_hi)

                        @pl.when(any_true(hitv))
                        def _splat():
                            gx = (pi & wmask).astype(jnp.float32)
                            x = (gx + fxb[sl]) * scale_v
                            xt = x.astype(jnp.int32)
                            x0 = xt - jnp.where(
                                xt.astype(jnp.float32) > x, 1, 0)
                            yt = y.astype(jnp.int32)
                            y0 = yt - jnp.where(
                                yt.astype(jnp.float32) > y, 1, 0)
                            wx1 = x - x0.astype(jnp.float32)
                            wx0 = 1.0 - wx1
                            wy1 = y - y0.astype(jnp.float32)
                            wy0 = 1.0 - wy1
                            vals = [cb[sl] for cb in cbufs]
                            ly = y0 - lo
                            x1 = x0 + 1
                            for lyv, wy in ((ly, wy0), (ly + 1, wy1)):
                                my = (lyv >= 0) & (lyv < rb)
                                rowb = lyv * ow
                                for xv, wx in ((x0, wx0), (x1, wx1)):
                                    m = my & (xv >= 0) & (xv < ow)
                                    wgt = wy * wx
                                    idx = jnp.where(m, rowb + xv, 0)
                                    for ch in range(chans):
                                        plsc.addupdate_scatter(
                                            acc, [idx + ch * rb * ow],
                                            vals[ch] * wgt, mask=m)
                        return c2
                    with scope("proc"):
                        lax.fori_loop(0, vpc, proc, 0)
                return c
            with scope("chunks"):
                lax.fori_loop(0, nchunk, chunk_loop, 0)

            obase = b * chans * oh * ow + lo * ow
            flushes = [
                pltpu.async_copy(
                    acc.at[pl.ds(ch * rb * ow, rb * ow)],
                    out_hbm.at[pl.ds(obase + ch * oh * ow, rb * ow)],
                    osem)
                for ch in range(chans)
            ]
            last = b == batch - 1
            with scope("flush_zero"):
                for ch in range(chans):
                    flushes[ch].wait()

                    @pl.when(jnp.logical_not(last))
                    def _rezero():
                        zero_range(ch * rb * ow, rb * ow)
            return carry

        lax.fori_loop(0, batch, per_batch, 0)

    mesh = plsc.VectorSubcoreMesh(core_axis_name="c", subcore_axis_name="s",
                                  num_cores=NC, num_subcores=NS)

    def wrapped(fy_hbm, fx_hbm, img_hbm, scl_hbm, out_hbm,
                acc, fyb, fxb, c0, c1, c2, sclb,
                minb, maxb, mint, maxt, shmin, shmax, sem, osem):
        return body(fy_hbm, fx_hbm, img_hbm, scl_hbm, out_hbm,
                    acc, fyb, fxb, [c0, c1, c2], sclb,
                    minb, maxb, mint, maxt, shmin, shmax, sem, osem)

    @jax.jit
    def warp(fy, fx, imgf, scl):
        return pl.kernel(
            wrapped,
            out_type=jax.ShapeDtypeStruct((batch * chans * oh * ow,),
                                          jnp.float32),
            mesh=mesh,
            interpret=interpret,
            compiler_params=pltpu.CompilerParams(needs_layout_passes=False),
            scratch_types=[
                pltpu.VMEM((band,), jnp.float32),
                pltpu.VMEM((chunk,), jnp.float32),
                pltpu.VMEM((chunk,), jnp.float32),
                pltpu.VMEM((chunk,), jnp.float32),
                pltpu.VMEM((chunk,), jnp.float32),
                pltpu.VMEM((chunk,), jnp.float32),
                pltpu.VMEM((L,), jnp.float32),
                pltpu.VMEM((ppt * L,), jnp.float32),
                pltpu.VMEM((ppt * L,), jnp.float32),
                pltpu.VMEM((npair * L,), jnp.float32),
                pltpu.VMEM((npair * L,), jnp.float32),
                pltpu.VMEM_SHARED((npair * L,), jnp.float32),
                pltpu.VMEM_SHARED((npair * L,), jnp.float32),
                pltpu.SemaphoreType.DMA,
                pltpu.SemaphoreType.DMA,
            ],
        )(fy, fx, imgf, scl)

    def run(img, flow, scale):
        fy = flow[:, 1, :, :].reshape(batch * n)
        fx = flow[:, 0, :, :].reshape(batch * n)
        imgf = img.reshape(batch * chans * n)
        scl = jnp.full((L,), scale, jnp.float32)
        out = warp(fy, fx, imgf, scl)
        return out.reshape(batch, chans, oh, ow)

    return run


_run = _build(8, 3, 256, 256, 2048)


def kernel(img, flow, scale):
    return _run(img, flow, scale)


# speedup vs baseline: 41.8628x; 1.3442x over previous
import functools

import jax
import jax.numpy as jnp
from jax import lax
from jax.experimental import pallas as pl
from jax.experimental.pallas import tpu as pltpu
from jax.experimental.pallas import tpu_sc as plsc

SCALE_CONST = 4
NC, NS, L = 2, 16, 16
NW = NC * NS


def _build(batch, chans, h, w, chunk, interpret=False):
    n = h * w
    oh, ow = h * SCALE_CONST, w * SCALE_CONST
    rb = oh // NW
    band = chans * rb * ow
    vpc = chunk // L
    nchunk = n // chunk
    npair = batch * nchunk
    ppt = npair // NS
    wshift = w.bit_length() - 1
    wmask = w - 1
    assert (1 << wshift) == w and n % chunk == 0 and oh % NW == 0
    assert npair % NS == 0
    jshift = nchunk.bit_length() - 1
    assert (1 << jshift) == nchunk

    def body(fy_hbm, fx_hbm, img_hbm, scl_hbm, out_hbm,
             acc, fyb, fxb, cbufs, sclb, minb, maxb, mint, maxt,
             shmin, shmax, sem, osem):
        cid = lax.axis_index("c")
        sid = lax.axis_index("s")
        wid = sid * NC + cid
        lo = wid * rb

        def any_true(mvec):
            return plsc.all_reduce_population_count(mvec)[0] > 0

        pltpu.sync_copy(scl_hbm, sclb)
        scale_v = sclb[...]

        iota = lax.iota(jnp.int32, L)
        lo_f = lo.astype(jnp.float32)
        band_lo = jnp.full((L,), lo_f - 1.0, jnp.float32)
        band_hi = jnp.full((L,), lo_f + rb, jnp.float32)
        zeros = jnp.zeros((L,), jnp.float32)

        scope = jax.named_scope

        def range_pair(i, c):
            k = sid * ppt + i
            b = k >> jshift
            j = k - (b << jshift)
            pltpu.sync_copy(fy_hbm.at[pl.ds(b * n + j * chunk, chunk)], fyb)

            def mm(v, carry):
                mn, mx = carry
                pi = j * chunk + v * L + iota
                gy = (pi >> wshift).astype(jnp.float32)
                y = (gy + fyb[pl.ds(v * L, L)]) * scale_v
                return jnp.minimum(mn, y), jnp.maximum(mx, y)
            mn, mx = lax.fori_loop(
                0, vpc, mm,
                (jnp.full((L,), 3.0e38, jnp.float32),
                 jnp.full((L,), -3.0e38, jnp.float32)))
            minb[pl.ds(i * L, L)] = mn
            maxb[pl.ds(i * L, L)] = mx
            return c
        with scope("phase0_range"):
            lax.fori_loop(0, ppt, range_pair, 0)
            pltpu.sync_copy(minb, shmin.at[pl.ds(sid * ppt * L, ppt * L)])
            pltpu.sync_copy(maxb, shmax.at[pl.ds(sid * ppt * L, ppt * L)])
            plsc.subcore_barrier()
            pltpu.sync_copy(shmin, mint)
            pltpu.sync_copy(shmax, maxt)

        def zero_rows(r0, nrows):
            def zloop(i, c):
                for k in range(ow // (8 * L)):
                    off = k * (8 * L)
                    for kk in range(8):
                        acc[r0 + i, pl.ds(off + kk * L, L)] = zeros
                return c
            lax.fori_loop(0, nrows, zloop, 0)

        with scope("zero_init"):
            zero_rows(0, chans * rb)

        def per_batch(b, carry):
            def chunk_loop(j, c):
                k16 = (b * nchunk + j) * L
                cmin = mint[pl.ds(k16, L)]
                cmax = maxt[pl.ds(k16, L)]
                hit = (cmax >= band_lo) & (cmin < band_hi)

                @pl.when(any_true(hit))
                def _process():
                    base = j * chunk
                    hbase = b * n + base
                    ibase = b * chans * n + base
                    cps = [
                        pltpu.async_copy(
                            fy_hbm.at[pl.ds(hbase, chunk)], fyb, sem),
                        pltpu.async_copy(
                            fx_hbm.at[pl.ds(hbase, chunk)], fxb, sem),
                    ] + [
                        pltpu.async_copy(
                            img_hbm.at[pl.ds(ibase + ch * n, chunk)],
                            cbufs[ch], sem)
                        for ch in range(chans)
                    ]
                    with scope("fetch_wait"):
                        for cp in cps:
                            cp.wait()

                    def proc(i, c2):
                        sl = pl.ds(i * L, L)
                        pi = base + i * L + iota
                        gy = (pi >> wshift).astype(jnp.float32)
                        y = (gy + fyb[sl]) * scale_v
                        hitv = (y >= band_lo) & (y < band_hi)

                        @pl.when(any_true(hitv))
                        def _splat():
                            gx = (pi & wmask).astype(jnp.float32)
                            x = (gx + fxb[sl]) * scale_v
                            xt = x.astype(jnp.int32)
                            x0 = xt - jnp.where(
                                xt.astype(jnp.float32) > x, 1, 0)
                            yt = y.astype(jnp.int32)
                            y0 = yt - jnp.where(
                                yt.astype(jnp.float32) > y, 1, 0)
                            wx1 = x - x0.astype(jnp.float32)
                            wx0 = 1.0 - wx1
                            wy1 = y - y0.astype(jnp.float32)
                            wy0 = 1.0 - wy1
                            vals = [cb[sl] for cb in cbufs]
                            ly = y0 - lo
                            x1 = x0 + 1
                            for lyv, wy in ((ly, wy0), (ly + 1, wy1)):
                                my = (lyv >= 0) & (lyv < rb)
                                for xv, wx in ((x0, wx0), (x1, wx1)):
                                    m = my & (xv >= 0) & (xv < ow)
                                    wgt = wy * wx
                                    rowi = jnp.where(m, lyv, 0)
                                    coli = jnp.where(m, xv, 0)
                                    for ch in range(chans):
                                        plsc.addupdate_scatter(
                                            acc, [rowi + ch * rb, coli],
                                            vals[ch] * wgt, mask=m)
                        return c2
                    with scope("proc"):
                        lax.fori_loop(0, vpc, proc, 0)
                return c
            with scope("chunks"):
                lax.fori_loop(0, nchunk, chunk_loop, 0)

            flushes = [
                pltpu.async_copy(
                    acc.at[pl.ds(ch * rb, rb), :],
                    out_hbm.at[b, ch, pl.ds(lo, rb), :],
                    osem)
                for ch in range(chans)
            ]
            last = b == batch - 1
            with scope("flush_zero"):
                for ch in range(chans):
                    flushes[ch].wait()

                    @pl.when(jnp.logical_not(last))
                    def _rezero():
                        zero_rows(ch * rb, rb)
            return carry

        lax.fori_loop(0, batch, per_batch, 0)

    mesh = plsc.VectorSubcoreMesh(core_axis_name="c", subcore_axis_name="s",
                                  num_cores=NC, num_subcores=NS)

    def wrapped(fy_hbm, fx_hbm, img_hbm, scl_hbm, out_hbm,
                acc, fyb, fxb, c0, c1, c2, sclb,
                minb, maxb, mint, maxt, shmin, shmax, sem, osem):
        return body(fy_hbm, fx_hbm, img_hbm, scl_hbm, out_hbm,
                    acc, fyb, fxb, [c0, c1, c2], sclb,
                    minb, maxb, mint, maxt, shmin, shmax, sem, osem)

    @jax.jit
    def warp(fy, fx, imgf, scl):
        return pl.kernel(
            wrapped,
            out_type=jax.ShapeDtypeStruct((batch, chans, oh, ow),
                                          jnp.float32),
            mesh=mesh,
            interpret=interpret,
            compiler_params=pltpu.CompilerParams(needs_layout_passes=False),
            scratch_types=[
                pltpu.VMEM((chans * rb, ow), jnp.float32),
                pltpu.VMEM((chunk,), jnp.float32),
                pltpu.VMEM((chunk,), jnp.float32),
                pltpu.VMEM((chunk,), jnp.float32),
                pltpu.VMEM((chunk,), jnp.float32),
                pltpu.VMEM((chunk,), jnp.float32),
                pltpu.VMEM((L,), jnp.float32),
                pltpu.VMEM((ppt * L,), jnp.float32),
                pltpu.VMEM((ppt * L,), jnp.float32),
                pltpu.VMEM((npair * L,), jnp.float32),
                pltpu.VMEM((npair * L,), jnp.float32),
                pltpu.VMEM_SHARED((npair * L,), jnp.float32),
                pltpu.VMEM_SHARED((npair * L,), jnp.float32),
                pltpu.SemaphoreType.DMA,
                pltpu.SemaphoreType.DMA,
            ],
        )(fy, fx, imgf, scl)

    def run(img, flow, scale):
        fy = flow[:, 1, :, :].reshape(batch * n)
        fx = flow[:, 0, :, :].reshape(batch * n)
        imgf = img.reshape(batch * chans * n)
        scl = jnp.full((L,), scale, jnp.float32)
        return warp(fy, fx, imgf, scl)

    return run


_run = _build(8, 3, 256, 256, 2048)


def kernel(img, flow, scale):
    return _run(img, flow, scale)


# speedup vs baseline: 42.1738x; 1.0074x over previous
import functools

import jax
import jax.numpy as jnp
from jax import lax
from jax.experimental import pallas as pl
from jax.experimental.pallas import tpu as pltpu
from jax.experimental.pallas import tpu_sc as plsc

SCALE_CONST = 4
NC, NS, L = 2, 16, 16
NW = NC * NS


def _build(batch, chans, h, w, crows, interpret=False):
    n = h * w
    chunk = crows * w
    oh, ow = h * SCALE_CONST, w * SCALE_CONST
    rb = oh // NW
    vpr = w // L
    nchunk = n // chunk
    npair = batch * nchunk
    ppt = npair // NS
    assert n % chunk == 0 and oh % NW == 0 and w % L == 0
    assert npair % NS == 0
    jshift = nchunk.bit_length() - 1
    assert (1 << jshift) == nchunk

    def body(img_hbm, flow_hbm, scl_hbm, out_hbm,
             acc, fyb, fxb, cbufs, sclb, minb, maxb, mint, maxt,
             shmin, shmax, sem, osem):
        cid = lax.axis_index("c")
        sid = lax.axis_index("s")
        wid = sid * NC + cid
        lo = wid * rb

        def any_true(mvec):
            return plsc.all_reduce_population_count(mvec)[0] > 0

        pltpu.sync_copy(scl_hbm, sclb)
        scale_v = sclb[...]

        iota_f = lax.iota(jnp.int32, L).astype(jnp.float32)
        lo_f = lo.astype(jnp.float32)
        band_lo = jnp.full((L,), lo_f - 1.0, jnp.float32)
        band_hi = jnp.full((L,), lo_f + rb, jnp.float32)
        zeros = jnp.zeros((L,), jnp.float32)
        scope = jax.named_scope

        def range_pair(i, c):
            k = sid * ppt + i
            b = k >> jshift
            j = k - (b << jshift)
            pltpu.sync_copy(
                flow_hbm.at[b, 1, pl.ds(j * crows, crows), :], fyb)

            def mm_row(r, carry):
                row_f = (j * crows + r).astype(jnp.float32)
                gy = jnp.full((L,), row_f, jnp.float32)

                def mm(v, rc):
                    mn, mx = rc
                    y = (gy + fyb[r, pl.ds(v * L, L)]) * scale_v
                    return jnp.minimum(mn, y), jnp.maximum(mx, y)
                return lax.fori_loop(0, vpr, mm, carry)
            mn, mx = lax.fori_loop(
                0, crows, mm_row,
                (jnp.full((L,), 3.0e38, jnp.float32),
                 jnp.full((L,), -3.0e38, jnp.float32)))
            minb[pl.ds(i * L, L)] = mn
            maxb[pl.ds(i * L, L)] = mx
            return c
        with scope("phase0_range"):
            lax.fori_loop(0, ppt, range_pair, 0)
            pltpu.sync_copy(minb, shmin.at[pl.ds(sid * ppt * L, ppt * L)])
            pltpu.sync_copy(maxb, shmax.at[pl.ds(sid * ppt * L, ppt * L)])
            plsc.subcore_barrier()
            pltpu.sync_copy(shmin, mint)
            pltpu.sync_copy(shmax, maxt)

        def zero_rows(r0, nrows):
            def zloop(i, c):
                for k in range(ow // (8 * L)):
                    off = k * (8 * L)
                    for kk in range(8):
                        acc[r0 + i, pl.ds(off + kk * L, L)] = zeros
                return c
            lax.fori_loop(0, nrows, zloop, 0)

        with scope("zero_init"):
            zero_rows(0, chans * rb)

        def per_batch(b, carry):
            def chunk_loop(j, c):
                k16 = (b * nchunk + j) * L
                cmin = mint[pl.ds(k16, L)]
                cmax = maxt[pl.ds(k16, L)]
                hit = (cmax >= band_lo) & (cmin < band_hi)

                @pl.when(any_true(hit))
                def _process():
                    rsl = pl.ds(j * crows, crows)
                    cps = [
                        pltpu.async_copy(
                            flow_hbm.at[b, 1, rsl, :], fyb, sem),
                        pltpu.async_copy(
                            flow_hbm.at[b, 0, rsl, :], fxb, sem),
                    ] + [
                        pltpu.async_copy(
                            img_hbm.at[b, ch, rsl, :], cbufs[ch], sem)
                        for ch in range(chans)
                    ]
                    with scope("fetch_wait"):
                        for cp in cps:
                            cp.wait()

                    def row_loop(r, cr):
                        row_f = (j * crows + r).astype(jnp.float32)
                        gy = jnp.full((L,), row_f, jnp.float32)

                        def proc(v, c2):
                            sl = pl.ds(v * L, L)
                            y = (gy + fyb[r, sl]) * scale_v
                            hitv = (y >= band_lo) & (y < band_hi)

                            @pl.when(any_true(hitv))
                            def _splat():
                                gx = iota_f + (v * L).astype(jnp.float32)
                                x = (gx + fxb[r, sl]) * scale_v
                                xt = x.astype(jnp.int32)
                                x0 = xt - jnp.where(
                                    xt.astype(jnp.float32) > x, 1, 0)
                                yt = y.astype(jnp.int32)
                                y0 = yt - jnp.where(
                                    yt.astype(jnp.float32) > y, 1, 0)
                                wx1 = x - x0.astype(jnp.float32)
                                wx0 = 1.0 - wx1
                                wy1 = y - y0.astype(jnp.float32)
                                wy0 = 1.0 - wy1
                                vals = [cb[r, sl] for cb in cbufs]
                                ly = y0 - lo
                                x1 = x0 + 1
                                for lyv, wy in ((ly, wy0), (ly + 1, wy1)):
                                    my = (lyv >= 0) & (lyv < rb)
                                    rowi = jnp.where(my, lyv, 0)
                                    for xv, wx in ((x0, wx0), (x1, wx1)):
                                        m = my & (xv >= 0) & (xv < ow)
                                        wgt = wy * wx
                                        coli = jnp.where(m, xv, 0)
                                        for ch in range(chans):
                                            plsc.addupdate_scatter(
                                                acc,
                                                [rowi + ch * rb, coli],
                                                vals[ch] * wgt, mask=m)
                            return c2
                        lax.fori_loop(0, vpr, proc, 0)
                        return cr
                    with scope("proc"):
                        lax.fori_loop(0, crows, row_loop, 0)
                return c
            with scope("chunks"):
                lax.fori_loop(0, nchunk, chunk_loop, 0)

            flushes = [
                pltpu.async_copy(
                    acc.at[pl.ds(ch * rb, rb), :],
                    out_hbm.at[b, ch, pl.ds(lo, rb), :],
                    osem)
                for ch in range(chans)
            ]
            last = b == batch - 1
            with scope("flush_zero"):
                for ch in range(chans):
                    flushes[ch].wait()

                    @pl.when(jnp.logical_not(last))
                    def _rezero():
                        zero_rows(ch * rb, rb)
            return carry

        lax.fori_loop(0, batch, per_batch, 0)

    mesh = plsc.VectorSubcoreMesh(core_axis_name="c", subcore_axis_name="s",
                                  num_cores=NC, num_subcores=NS)

    def wrapped(img_hbm, flow_hbm, scl_hbm, out_hbm,
                acc, fyb, fxb, c0, c1, c2, sclb,
                minb, maxb, mint, maxt, shmin, shmax, sem, osem):
        return body(img_hbm, flow_hbm, scl_hbm, out_hbm,
                    acc, fyb, fxb, [c0, c1, c2], sclb,
                    minb, maxb, mint, maxt, shmin, shmax, sem, osem)

    @jax.jit
    def warp(img, flow, scl):
        return pl.kernel(
            wrapped,
            out_type=jax.ShapeDtypeStruct((batch, chans, oh, ow),
                                          jnp.float32),
            mesh=mesh,
            interpret=interpret,
            compiler_params=pltpu.CompilerParams(needs_layout_passes=False),
            scratch_types=[
                pltpu.VMEM((chans * rb, ow), jnp.float32),
                pltpu.VMEM((crows, w), jnp.float32),
                pltpu.VMEM((crows, w), jnp.float32),
                pltpu.VMEM((crows, w), jnp.float32),
                pltpu.VMEM((crows, w), jnp.float32),
                pltpu.VMEM((crows, w), jnp.float32),
                pltpu.VMEM((L,), jnp.float32),
                pltpu.VMEM((ppt * L,), jnp.float32),
                pltpu.VMEM((ppt * L,), jnp.float32),
                pltpu.VMEM((npair * L,), jnp.float32),
                pltpu.VMEM((npair * L,), jnp.float32),
                pltpu.VMEM_SHARED((npair * L,), jnp.float32),
                pltpu.VMEM_SHARED((npair * L,), jnp.float32),
                pltpu.SemaphoreType.DMA,
                pltpu.SemaphoreType.DMA,
            ],
        )(img, flow, scl)

    def run(img, flow, scale):
        scl = jnp.full((L,), scale, jnp.float32)
        return warp(img, flow, scl)

    return run


_run = _build(8, 3, 256, 256, 8)


def kernel(img, flow, scale):
    return _run(img, flow, scale)


# speedup vs baseline: 47.5819x; 1.1282x over previous
import functools

import jax
import jax.numpy as jnp
from jax import lax
from jax.experimental import pallas as pl
from jax.experimental.pallas import tpu as pltpu
from jax.experimental.pallas import tpu_sc as plsc

SCALE_CONST = 4
NC, NS, L = 2, 16, 16
NW = NC * NS


def _build(batch, chans, h, w, crows, interpret=False):
    n = h * w
    chunk = crows * w
    oh, ow = h * SCALE_CONST, w * SCALE_CONST
    rb = oh // NW
    vpr = w // L
    nchunk = n // chunk
    npair = batch * nchunk
    ppt = npair // NS
    assert n % chunk == 0 and oh % NW == 0 and w % L == 0
    assert npair % NS == 0
    jshift = nchunk.bit_length() - 1
    assert (1 << jshift) == nchunk

    def body(img_hbm, flow_hbm, scl_hbm, out_hbm,
             acc, fyb, fxb, cbufs, sclb, minb, maxb, mint, maxt,
             shmin, shmax, sem, osem):
        cid = lax.axis_index("c")
        sid = lax.axis_index("s")
        wid = sid * NC + cid
        lo = wid * rb

        def any_true(mvec):
            return plsc.all_reduce_population_count(mvec)[0] > 0

        pltpu.sync_copy(scl_hbm, sclb)
        scale_v = sclb[...]

        iota_f = lax.iota(jnp.int32, L).astype(jnp.float32)
        lo_f = lo.astype(jnp.float32)
        band_lo = jnp.full((L,), lo_f - 1.0, jnp.float32)
        band_hi = jnp.full((L,), lo_f + rb, jnp.float32)
        zeros = jnp.zeros((L,), jnp.float32)
        scope = jax.named_scope

        def range_pair(i, c):
            k = sid * ppt + i
            b = k >> jshift
            j = k - (b << jshift)
            pltpu.sync_copy(
                flow_hbm.at[b, 1, pl.ds(j * crows, crows), :], fyb)

            def mm_row(r, carry):
                row_f = (j * crows + r).astype(jnp.float32)
                gy = jnp.full((L,), row_f, jnp.float32)

                def mm(v, rc):
                    mn, mx = rc
                    y = (gy + fyb[r, pl.ds(v * L, L)]) * scale_v
                    return jnp.minimum(mn, y), jnp.maximum(mx, y)
                return lax.fori_loop(0, vpr, mm, carry)
            mn, mx = lax.fori_loop(
                0, crows, mm_row,
                (jnp.full((L,), 3.0e38, jnp.float32),
                 jnp.full((L,), -3.0e38, jnp.float32)))
            minb[pl.ds(i * L, L)] = mn
            maxb[pl.ds(i * L, L)] = mx
            return c
        with scope("phase0_range"):
            lax.fori_loop(0, ppt, range_pair, 0)
            pltpu.sync_copy(minb, shmin.at[pl.ds(sid * ppt * L, ppt * L)])
            pltpu.sync_copy(maxb, shmax.at[pl.ds(sid * ppt * L, ppt * L)])
            plsc.subcore_barrier()
            pltpu.sync_copy(shmin, mint)
            pltpu.sync_copy(shmax, maxt)

        def zero_rows(r0, nrows):
            def zloop(i, c):
                for k in range(ow // (8 * L)):
                    off = k * (8 * L)
                    for kk in range(8):
                        acc[r0 + i, pl.ds(off + kk * L, L)] = zeros
                return c
            lax.fori_loop(0, nrows, zloop, 0)

        with scope("zero_init"):
            zero_rows(0, chans * rb)

        def per_batch(b, carry):
            def chunk_loop(j, c):
                k16 = (b * nchunk + j) * L
                cmin = mint[pl.ds(k16, L)]
                cmax = maxt[pl.ds(k16, L)]
                hit = (cmax >= band_lo) & (cmin < band_hi)

                @pl.when(any_true(hit))
                def _process():
                    rsl = pl.ds(j * crows, crows)
                    cps = [
                        pltpu.async_copy(
                            flow_hbm.at[b, 1, rsl, :], fyb, sem),
                        pltpu.async_copy(
                            flow_hbm.at[b, 0, rsl, :], fxb, sem),
                    ] + [
                        pltpu.async_copy(
                            img_hbm.at[b, ch, rsl, :], cbufs[ch], sem)
                        for ch in range(chans)
                    ]
                    with scope("fetch_wait"):
                        for cp in cps:
                            cp.wait()

                    def row_loop(r, cr):
                        row_f = (j * crows + r).astype(jnp.float32)
                        gy = jnp.full((L,), row_f, jnp.float32)

                        def proc(v, c2):
                            sl = pl.ds(v * L, L)
                            y = (gy + fyb[r, sl]) * scale_v
                            gx = iota_f + (v * L).astype(jnp.float32)
                            x = (gx + fxb[r, sl]) * scale_v
                            xt = x.astype(jnp.int32)
                            x0 = xt - jnp.where(
                                xt.astype(jnp.float32) > x, 1, 0)
                            yt = y.astype(jnp.int32)
                            y0 = yt - jnp.where(
                                yt.astype(jnp.float32) > y, 1, 0)
                            wx1 = x - x0.astype(jnp.float32)
                            wx0 = 1.0 - wx1
                            wy1 = y - y0.astype(jnp.float32)
                            wy0 = 1.0 - wy1
                            vals = [cb[r, sl] for cb in cbufs]
                            ly = y0 - lo
                            x1 = x0 + 1
                            for lyv, wy in ((ly, wy0), (ly + 1, wy1)):
                                my = (lyv >= 0) & (lyv < rb)
                                rowi = jnp.where(my, lyv, 0)
                                for xv, wx in ((x0, wx0), (x1, wx1)):
                                    m = my & (xv >= 0) & (xv < ow)
                                    wgt = wy * wx
                                    coli = jnp.where(m, xv, 0)
                                    for ch in range(chans):
                                        plsc.addupdate_scatter(
                                            acc,
                                            [rowi + ch * rb, coli],
                                            vals[ch] * wgt, mask=m)
                            return c2
                        lax.fori_loop(0, vpr, proc, 0)
                        return cr
                    with scope("proc"):
                        lax.fori_loop(0, crows, row_loop, 0)
                return c
            with scope("chunks"):
                lax.fori_loop(0, nchunk, chunk_loop, 0)

            flushes = [
                pltpu.async_copy(
                    acc.at[pl.ds(ch * rb, rb), :],
                    out_hbm.at[b, ch, pl.ds(lo, rb), :],
                    osem)
                for ch in range(chans)
            ]
            last = b == batch - 1
            with scope("flush_zero"):
                for ch in range(chans):
                    flushes[ch].wait()

                    @pl.when(jnp.logical_not(last))
                    def _rezero():
                        zero_rows(ch * rb, rb)
            return carry

        lax.fori_loop(0, batch, per_batch, 0)

    mesh = plsc.VectorSubcoreMesh(core_axis_name="c", subcore_axis_name="s",
                                  num_cores=NC, num_subcores=NS)

    def wrapped(img_hbm, flow_hbm, scl_hbm, out_hbm,
                acc, fyb, fxb, c0, c1, c2, sclb,
                minb, maxb, mint, maxt, shmin, shmax, sem, osem):
        return body(img_hbm, flow_hbm, scl_hbm, out_hbm,
                    acc, fyb, fxb, [c0, c1, c2], sclb,
                    minb, maxb, mint, maxt, shmin, shmax, sem, osem)

    @jax.jit
    def warp(img, flow, scl):
        return pl.kernel(
            wrapped,
            out_type=jax.ShapeDtypeStruct((batch, chans, oh, ow),
                                          jnp.float32),
            mesh=mesh,
            interpret=interpret,
            compiler_params=pltpu.CompilerParams(needs_layout_passes=False),
            scratch_types=[
                pltpu.VMEM((chans * rb, ow), jnp.float32),
                pltpu.VMEM((crows, w), jnp.float32),
                pltpu.VMEM((crows, w), jnp.float32),
                pltpu.VMEM((crows, w), jnp.float32),
                pltpu.VMEM((crows, w), jnp.float32),
                pltpu.VMEM((crows, w), jnp.float32),
                pltpu.VMEM((L,), jnp.float32),
                pltpu.VMEM((ppt * L,), jnp.float32),
                pltpu.VMEM((ppt * L,), jnp.float32),
                pltpu.VMEM((npair * L,), jnp.float32),
                pltpu.VMEM((npair * L,), jnp.float32),
                pltpu.VMEM_SHARED((npair * L,), jnp.float32),
                pltpu.VMEM_SHARED((npair * L,), jnp.float32),
                pltpu.SemaphoreType.DMA,
                pltpu.SemaphoreType.DMA,
            ],
        )(img, flow, scl)

    def run(img, flow, scale):
        scl = jnp.full((L,), scale, jnp.float32)
        return warp(img, flow, scl)

    return run


_run = _build(8, 3, 256, 256, 8)


def kernel(img, flow, scale):
    return _run(img, flow, scale)


# speedup vs baseline: 54.3406x; 1.1420x over previous
import functools

import jax
import jax.numpy as jnp
from jax import lax
from jax.experimental import pallas as pl
from jax.experimental.pallas import tpu as pltpu
from jax.experimental.pallas import tpu_sc as plsc

SCALE_CONST = 4
NC, NS, L = 2, 16, 16
NW = NC * NS


def _build(batch, chans, h, w, crows, interpret=False):
    n = h * w
    chunk = crows * w
    oh, ow = h * SCALE_CONST, w * SCALE_CONST
    rb = oh // NW
    vpr = w // L
    nchunk = n // chunk
    npair = batch * nchunk
    ppt = npair // NS
    assert n % chunk == 0 and oh % NW == 0 and w % L == 0
    assert npair % NS == 0
    jshift = nchunk.bit_length() - 1
    assert (1 << jshift) == nchunk

    def body(img_hbm, flow_hbm, scl_hbm, out_hbm,
             acc, fyb, fxb, cbufs, sclb, minb, maxb, mint, maxt,
             shmin, shmax, sem, osem):
        cid = lax.axis_index("c")
        sid = lax.axis_index("s")
        wid = sid * NC + cid
        lo = wid * rb

        def any_true(mvec):
            return plsc.all_reduce_population_count(mvec)[0] > 0

        pltpu.sync_copy(scl_hbm, sclb)
        scale_v = sclb[...]

        iota_f = lax.iota(jnp.int32, L).astype(jnp.float32)
        lo_f = lo.astype(jnp.float32)
        band_lo = jnp.full((L,), lo_f - 1.0, jnp.float32)
        band_hi = jnp.full((L,), lo_f + rb, jnp.float32)
        zeros = jnp.zeros((L,), jnp.float32)
        scope = jax.named_scope

        def range_pair(i, c):
            k = sid * ppt + i
            b = k >> jshift
            j = k - (b << jshift)
            pltpu.sync_copy(
                flow_hbm.at[b, 1, pl.ds(j * crows, crows), :], fyb)

            def mm_row(r, carry):
                row_f = (j * crows + r).astype(jnp.float32)
                gy = jnp.full((L,), row_f, jnp.float32)

                def mm(v, rc):
                    mn, mx = rc
                    y = (gy + fyb[r, pl.ds(v * L, L)]) * scale_v
                    return jnp.minimum(mn, y), jnp.maximum(mx, y)
                return lax.fori_loop(0, vpr, mm, carry)
            mn, mx = lax.fori_loop(
                0, crows, mm_row,
                (jnp.full((L,), 3.0e38, jnp.float32),
                 jnp.full((L,), -3.0e38, jnp.float32)))
            minb[pl.ds(i * L, L)] = mn
            maxb[pl.ds(i * L, L)] = mx
            return c
        with scope("phase0_range"):
            lax.fori_loop(0, ppt, range_pair, 0)
            pltpu.sync_copy(minb, shmin.at[pl.ds(sid * ppt * L, ppt * L)])
            pltpu.sync_copy(maxb, shmax.at[pl.ds(sid * ppt * L, ppt * L)])
            plsc.subcore_barrier()
            pltpu.sync_copy(shmin, mint)
            pltpu.sync_copy(shmax, maxt)

        def zero_rows(r0, nrows):
            def zloop(i, c):
                for k in range(ow // (8 * L)):
                    off = k * (8 * L)
                    for kk in range(8):
                        acc[r0 + i, pl.ds(off + kk * L, L)] = zeros
                return c
            lax.fori_loop(0, nrows, zloop, 0)

        with scope("zero_init"):
            zero_rows(0, chans * rb)

        def per_batch(b, carry):
            def chunk_loop(j, c):
                k16 = (b * nchunk + j) * L
                cmin = mint[pl.ds(k16, L)]
                cmax = maxt[pl.ds(k16, L)]
                hit = (cmax >= band_lo) & (cmin < band_hi)

                @pl.when(any_true(hit))
                def _process():
                    rsl = pl.ds(j * crows, crows)
                    cps = [
                        pltpu.async_copy(
                            flow_hbm.at[b, 1, rsl, :], fyb, sem),
                        pltpu.async_copy(
                            flow_hbm.at[b, 0, rsl, :], fxb, sem),
                    ] + [
                        pltpu.async_copy(
                            img_hbm.at[b, ch, rsl, :], cbufs[ch], sem)
                        for ch in range(chans)
                    ]
                    with scope("fetch_wait"):
                        for cp in cps:
                            cp.wait()

                    def row_loop(r, cr):
                        row_f = (j * crows + r).astype(jnp.float32)
                        gy = jnp.full((L,), row_f, jnp.float32)

                        def proc(v):
                            sl = pl.ds(v * L, L)
                            y = (gy + fyb[r, sl]) * scale_v
                            gx = iota_f + (v * L).astype(jnp.float32)
                            x = (gx + fxb[r, sl]) * scale_v
                            xt = x.astype(jnp.int32)
                            x0 = xt - jnp.where(
                                xt.astype(jnp.float32) > x, 1, 0)
                            yt = y.astype(jnp.int32)
                            y0 = yt - jnp.where(
                                yt.astype(jnp.float32) > y, 1, 0)
                            wx1 = x - x0.astype(jnp.float32)
                            wx0 = 1.0 - wx1
                            wy1 = y - y0.astype(jnp.float32)
                            wy0 = 1.0 - wy1
                            vals = [cb[r, sl] for cb in cbufs]
                            ly = y0 - lo
                            x1 = x0 + 1
                            for lyv, wy in ((ly, wy0), (ly + 1, wy1)):
                                my = (lyv >= 0) & (lyv < rb)
                                rowi = jnp.where(my, lyv, 0)
                                for xv, wx in ((x0, wx0), (x1, wx1)):
                                    m = my & (xv >= 0) & (xv < ow)
                                    wgt = wy * wx
                                    coli = jnp.where(m, xv, 0)
                                    for ch in range(chans):
                                        plsc.addupdate_scatter(
                                            acc,
                                            [rowi + ch * rb, coli],
                                            vals[ch] * wgt, mask=m)
                        plsc.parallel_loop(0, vpr, unroll=2)(proc)
                        return cr
                    with scope("proc"):
                        lax.fori_loop(0, crows, row_loop, 0)
                return c
            with scope("chunks"):
                lax.fori_loop(0, nchunk, chunk_loop, 0)

            flushes = [
                pltpu.async_copy(
                    acc.at[pl.ds(ch * rb, rb), :],
                    out_hbm.at[b, ch, pl.ds(lo, rb), :],
                    osem)
                for ch in range(chans)
            ]
            last = b == batch - 1
            with scope("flush_zero"):
                for ch in range(chans):
                    flushes[ch].wait()

                    @pl.when(jnp.logical_not(last))
                    def _rezero():
                        zero_rows(ch * rb, rb)
            return carry

        lax.fori_loop(0, batch, per_batch, 0)

    mesh = plsc.VectorSubcoreMesh(core_axis_name="c", subcore_axis_name="s",
                                  num_cores=NC, num_subcores=NS)

    def wrapped(img_hbm, flow_hbm, scl_hbm, out_hbm,
                acc, fyb, fxb, c0, c1, c2, sclb,
                minb, maxb, mint, maxt, shmin, shmax, sem, osem):
        return body(img_hbm, flow_hbm, scl_hbm, out_hbm,
                    acc, fyb, fxb, [c0, c1, c2], sclb,
                    minb, maxb, mint, maxt, shmin, shmax, sem, osem)

    @jax.jit
    def warp(img, flow, scl):
        return pl.kernel(
            wrapped,
            out_type=jax.ShapeDtypeStruct((batch, chans, oh, ow),
                                          jnp.float32),
            mesh=mesh,
            interpret=interpret,
            compiler_params=pltpu.CompilerParams(needs_layout_passes=False),
            scratch_types=[
                pltpu.VMEM((chans * rb, ow), jnp.float32),
                pltpu.VMEM((crows, w), jnp.float32),
                pltpu.VMEM((crows, w), jnp.float32),
                pltpu.VMEM((crows, w), jnp.float32),
                pltpu.VMEM((crows, w), jnp.float32),
                pltpu.VMEM((crows, w), jnp.float32),
                pltpu.VMEM((L,), jnp.float32),
                pltpu.VMEM((ppt * L,), jnp.float32),
                pltpu.VMEM((ppt * L,), jnp.float32),
                pltpu.VMEM((npair * L,), jnp.float32),
                pltpu.VMEM((npair * L,), jnp.float32),
                pltpu.VMEM_SHARED((npair * L,), jnp.float32),
                pltpu.VMEM_SHARED((npair * L,), jnp.float32),
                pltpu.SemaphoreType.DMA,
                pltpu.SemaphoreType.DMA,
            ],
        )(img, flow, scl)

    def run(img, flow, scale):
        scl = jnp.full((L,), scale, jnp.float32)
        return warp(img, flow, scl)

    return run


_run = _build(8, 3, 256, 256, 8)


def kernel(img, flow, scale):
    return _run(img, flow, scale)
